# baseline (speedup 1.0000x reference)
import functools

import jax
import jax.numpy as jnp
from jax import lax
from jax.experimental import pallas as pl
from jax.experimental.pallas import tpu as pltpu
from jax.experimental.pallas import tpu_sc as plsc

N = 20000
NPAD = 20480
SUBS = 16
LANES = 16
E = NPAD // SUBS
CHUNKS = E // LANES
KEEP = 100
OUTROWS = 104
REC = 2 * LANES
IMG = 1024.0
NEG = -1e9
PAD_NEG = -3e9
BIG = 3.0e38


def _build_rec(m, g, lane, x1v, y1v, x2v, y2v, arv, base):
    mv = jnp.full((LANES,), m, jnp.float32)
    gv = jnp.full((LANES,), g, jnp.float32)
    iv = jnp.full((LANES,), g.astype(jnp.int32) - base, jnp.int32)
    x1w = plsc.load_gather(x1v, [iv])
    y1w = plsc.load_gather(y1v, [iv])
    x2w = plsc.load_gather(x2v, [iv])
    y2w = plsc.load_gather(y2v, [iv])
    arw = plsc.load_gather(arv, [iv])
    return jnp.where(
        lane == 0, x1w,
        jnp.where(lane == 1, y1w,
                  jnp.where(lane == 2, x2w,
                            jnp.where(lane == 3, y2w,
                                      jnp.where(lane == 4, mv,
                                                jnp.where(lane == 5, arw, gv))))))


def _build_cands(m1, g1, m2, g2, lane, x1v, y1v, x2v, y2v, arv, base):
    M1 = jnp.max(m1)
    G1 = jnp.min(jnp.where(m1 == jnp.full((LANES,), M1, jnp.float32), g1, BIG))
    winlane = (m1 == jnp.full((LANES,), M1, jnp.float32)) & (
        g1 == jnp.full((LANES,), G1, jnp.float32))
    cv = jnp.where(winlane, m2, m1)
    cg = jnp.where(winlane, g2, g1)
    M2 = jnp.max(cv)
    G2 = jnp.min(jnp.where(cv == jnp.full((LANES,), M2, jnp.float32), cg, BIG))
    rec1 = _build_rec(M1, G1, lane, x1v, y1v, x2v, y2v, arv, base)
    rec2 = _build_rec(M2, G2, lane, x1v, y1v, x2v, y2v, arv, base)
    return rec1, rec2


def _nms_body(bx1_h, by1_h, bx2_h, by2_h, dx_h, dy_h, dw_h, dh_h, sc_h,
              out_h,
              x1v, y1v, x2v, y2v, dxv, dyv, dwv, dhv, arv, wkv,
              candv, allc, outv, shared):
    sub = lax.axis_index("s")
    core = lax.axis_index("c")
    base = sub * E
    lane = lax.broadcasted_iota(jnp.int32, (LANES,), 0)
    lanef = lane.astype(jnp.float32)
    basef = jnp.float32(E) * sub.astype(jnp.float32)
    top0 = (jnp.full((LANES,), -BIG, jnp.float32),
            jnp.zeros((LANES,), jnp.float32),
            jnp.full((LANES,), -BIG, jnp.float32),
            jnp.zeros((LANES,), jnp.float32))

    pltpu.sync_copy(bx1_h.at[pl.ds(base, E)], x1v)
    pltpu.sync_copy(by1_h.at[pl.ds(base, E)], y1v)
    pltpu.sync_copy(bx2_h.at[pl.ds(base, E)], x2v)
    pltpu.sync_copy(by2_h.at[pl.ds(base, E)], y2v)
    pltpu.sync_copy(dx_h.at[pl.ds(base, E)], dxv)
    pltpu.sync_copy(dy_h.at[pl.ds(base, E)], dyv)
    pltpu.sync_copy(dw_h.at[pl.ds(base, E)], dwv)
    pltpu.sync_copy(dh_h.at[pl.ds(base, E)], dhv)
    pltpu.sync_copy(sc_h.at[pl.ds(base, E)], wkv)

    def track2(nwk, gidf, m1, g1, m2, g2):
        gt1 = nwk > m1
        gt2 = nwk > m2
        m2n = jnp.where(gt1, m1, jnp.where(gt2, nwk, m2))
        g2n = jnp.where(gt1, g1, jnp.where(gt2, gidf, g2))
        return (jnp.where(gt1, nwk, m1), jnp.where(gt1, gidf, g1), m2n, g2n)

    def decode_chunk(j, carry):
        m1, g1, m2, g2 = carry
        sl = pl.ds(j * LANES, LANES)
        x1 = x1v[sl]
        y1 = y1v[sl]
        x2 = x2v[sl]
        y2 = y2v[sl]
        w = x2 - x1 + 1.0
        h = y2 - y1 + 1.0
        cx = x1 + 0.5 * w
        cy = y1 + 0.5 * h
        pcx = dxv[sl] * w + cx
        pcy = dyv[sl] * h + cy
        pw = jnp.exp(dwv[sl]) * w
        ph = jnp.exp(dhv[sl]) * h
        nx1 = jnp.clip(pcx - 0.5 * pw, 0.0, IMG)
        ny1 = jnp.clip(pcy - 0.5 * ph, 0.0, IMG)
        nx2 = jnp.clip(pcx + 0.5 * pw, 0.0, IMG)
        ny2 = jnp.clip(pcy + 0.5 * ph, 0.0, IMG)
        ar = jnp.maximum(nx2 - nx1, 0.0) * jnp.maximum(ny2 - ny1, 0.0)
        x1v[sl] = nx1
        y1v[sl] = ny1
        x2v[sl] = nx2
        y2v[sl] = ny2
        arv[sl] = ar
        gidf = basef + (j * LANES).astype(jnp.float32) + lanef
        return track2(wkv[sl], gidf, m1, g1, m2, g2)

    m1, g1, m2, g2 = lax.fori_loop(0, CHUNKS, decode_chunk, top0)
    rec10, rec20 = _build_cands(m1, g1, m2, g2, lane,
                                x1v, y1v, x2v, y2v, arv, base)

    def cond(carry):
        return carry[0] < KEEP

    def step(carry):
        t, e, rec1, rec2 = carry
        p = lax.bitwise_and(e, 1)
        candv[pl.ds(0, LANES)] = rec1
        candv[pl.ds(LANES, LANES)] = rec2
        pltpu.sync_copy(
            candv, shared.at[pl.ds(p * (SUBS * REC) + sub * REC, REC)])
        plsc.subcore_barrier()
        pltpu.sync_copy(shared.at[pl.ds(p * (SUBS * REC), SUBS * REC)], allc)
        vals1 = plsc.load_gather(allc, [lane * REC + 4])
        gids1 = plsc.load_gather(allc, [lane * REC + 6])
        vals2 = plsc.load_gather(allc, [lane * REC + LANES + 4])
        gids2 = plsc.load_gather(allc, [lane * REC + LANES + 6])
        M1 = jnp.max(vals1)
        tie1 = vals1 == jnp.full((LANES,), M1, jnp.float32)
        G1 = jnp.min(jnp.where(tie1, gids1, BIG))
        rowc1 = jnp.where(tie1 & (gids1 == jnp.full((LANES,), G1, jnp.float32)),
                          lane, 9999)
        r1 = jnp.min(rowc1)
        rb1 = r1 * REC
        win1 = plsc.load_gather(allc, [jnp.full((LANES,), rb1, jnp.int32) + lane])
        iw_mask = lane == jnp.full((LANES,), r1, jnp.int32)
        cv = jnp.where(iw_mask, vals2, vals1)
        cg = jnp.where(iw_mask, gids2, gids1)
        M2 = jnp.max(cv)
        tie2 = cv == jnp.full((LANES,), M2, jnp.float32)
        G2 = jnp.min(jnp.where(tie2, cg, BIG))
        rowc2 = jnp.where(tie2 & (cg == jnp.full((LANES,), G2, jnp.float32)),
                          lane, 9999)
        r2 = jnp.min(rowc2)
        rb2 = r2 * REC + jnp.where(r2 == r1, LANES, 0)
        win2 = plsc.load_gather(allc, [jnp.full((LANES,), rb2, jnp.int32) + lane])

        bx1 = plsc.load_gather(allc, [jnp.full((LANES,), rb1, jnp.int32)])
        by1 = plsc.load_gather(allc, [jnp.full((LANES,), rb1 + 1, jnp.int32)])
        bx2 = plsc.load_gather(allc, [jnp.full((LANES,), rb1 + 2, jnp.int32)])
        by2 = plsc.load_gather(allc, [jnp.full((LANES,), rb1 + 3, jnp.int32)])
        bar = plsc.load_gather(allc, [jnp.full((LANES,), rb1 + 5, jnp.int32)])
        cx1 = plsc.load_gather(allc, [jnp.full((LANES,), rb2, jnp.int32)])
        cy1 = plsc.load_gather(allc, [jnp.full((LANES,), rb2 + 1, jnp.int32)])
        cx2 = plsc.load_gather(allc, [jnp.full((LANES,), rb2 + 2, jnp.int32)])
        cy2 = plsc.load_gather(allc, [jnp.full((LANES,), rb2 + 3, jnp.int32)])
        car = plsc.load_gather(allc, [jnp.full((LANES,), rb2 + 5, jnp.int32)])

        sxx1 = jnp.maximum(bx1, cx1)
        syy1 = jnp.maximum(by1, cy1)
        sxx2 = jnp.minimum(bx2, cx2)
        syy2 = jnp.minimum(by2, cy2)
        sinter = jnp.maximum(sxx2 - sxx1, 0.0) * jnp.maximum(syy2 - syy1, 0.0)
        siou = sinter / (bar + car - sinter + 1e-9)
        specv = (~(siou > 0.5)) & (jnp.full((LANES,), M2, jnp.float32) > NEG)
        spec_i = jnp.max(jnp.where(specv, 1, 0))
        spec_b = spec_i > 0

        ok0 = (sub == 0) & (core == 0)

        @pl.when(ok0)
        def _():
            outv[pl.ds(t * LANES, LANES)] = win1

        @pl.when(ok0 & spec_b)
        def _():
            outv[pl.ds((t + 1) * LANES, LANES)] = win2

        wg1 = G1.astype(jnp.int32)
        wg2 = G2.astype(jnp.int32)

        @pl.when((wg1 >= base) & (wg1 < base + E))
        def _():
            plsc.store_scatter(wkv, [jnp.full((LANES,), wg1 - base, jnp.int32)],
                               jnp.full((LANES,), NEG, jnp.float32),
                               mask=lane == 0)

        @pl.when(spec_b & (wg2 >= base) & (wg2 < base + E))
        def _():
            plsc.store_scatter(wkv, [jnp.full((LANES,), wg2 - base, jnp.int32)],
                               jnp.full((LANES,), NEG, jnp.float32),
                               mask=lane == 0)

        cx1p = jnp.where(specv, cx1, -5000.0)
        cy1p = jnp.where(specv, cy1, -5000.0)
        cx2p = jnp.where(specv, cx2, -4000.0)
        cy2p = jnp.where(specv, cy2, -4000.0)

        def chunk(jj, carry):
            m1, g1, m2, g2 = carry
            for u in range(2):
                j = jj * 2 + u
                sl = pl.ds(j * LANES, LANES)
                x1 = x1v[sl]
                y1 = y1v[sl]
                x2 = x2v[sl]
                y2 = y2v[sl]
                ar = arv[sl]
                wk = wkv[sl]
                xx1 = jnp.maximum(bx1, x1)
                yy1 = jnp.maximum(by1, y1)
                xx2 = jnp.minimum(bx2, x2)
                yy2 = jnp.minimum(by2, y2)
                inter = jnp.maximum(xx2 - xx1, 0.0) * jnp.maximum(yy2 - yy1, 0.0)
                iou1 = inter / (bar + ar - inter + 1e-9)
                qx1 = jnp.maximum(cx1p, x1)
                qy1 = jnp.maximum(cy1p, y1)
                qx2 = jnp.minimum(cx2p, x2)
                qy2 = jnp.minimum(cy2p, y2)
                qint = jnp.maximum(qx2 - qx1, 0.0) * jnp.maximum(qy2 - qy1, 0.0)
                iou2 = qint / (car + ar - qint + 1e-9)
                sup = (iou1 > 0.5) | (iou2 > 0.5)
                nwk = jnp.where(sup, NEG, wk)
                wkv[sl] = nwk
                gidf = basef + (j * LANES).astype(jnp.float32) + lanef
                m1, g1, m2, g2 = track2(nwk, gidf, m1, g1, m2, g2)
            return (m1, g1, m2, g2)

        m1, g1, m2, g2 = lax.fori_loop(0, CHUNKS // 2, chunk, top0)
        rec1n, rec2n = _build_cands(m1, g1, m2, g2, lane,
                                    x1v, y1v, x2v, y2v, arv, base)
        return (t + 1 + spec_i, e + 1, rec1n, rec2n)

    lax.while_loop(cond, step, (jnp.int32(0), jnp.int32(0), rec10, rec20))

    @pl.when((sub == 0) & (core == 0))
    def _():
        pltpu.sync_copy(outv, out_h)


_SCRATCH = (
    [pltpu.VMEM((E,), jnp.float32)] * 10
    + [
        pltpu.VMEM((REC,), jnp.float32),
        pltpu.VMEM((SUBS * REC,), jnp.float32),
        pltpu.VMEM((OUTROWS * LANES,), jnp.float32),
        pltpu.VMEM_SHARED((2 * SUBS * REC,), jnp.float32),
    ]
)


@functools.lru_cache(maxsize=None)
def _get_nms():
    return functools.partial(
        pl.kernel,
        out_type=jax.ShapeDtypeStruct((OUTROWS * LANES,), jnp.float32),
        mesh=plsc.VectorSubcoreMesh(core_axis_name="c", subcore_axis_name="s"),
        compiler_params=pltpu.CompilerParams(needs_layout_passes=False),
        scratch_types=list(_SCRATCH),
    )(_nms_body)


def kernel(boxes, scores, box_deltas):
    pad = NPAD - N
    bx1 = jnp.pad(boxes[:, 0], (0, pad))
    by1 = jnp.pad(boxes[:, 1], (0, pad))
    bx2 = jnp.pad(boxes[:, 2], (0, pad))
    by2 = jnp.pad(boxes[:, 3], (0, pad))
    dx = jnp.pad(box_deltas[:, 0], (0, pad))
    dy = jnp.pad(box_deltas[:, 1], (0, pad))
    dw = jnp.pad(box_deltas[:, 2], (0, pad))
    dh = jnp.pad(box_deltas[:, 3], (0, pad))
    sc = jnp.pad(scores, (0, pad), constant_values=PAD_NEG)
    flat = _get_nms()(bx1, by1, bx2, by2, dx, dy, dw, dh, sc)
    return flat.reshape(OUTROWS, LANES)[:KEEP, :5]


# speedup vs baseline: 1.0234x; 1.0234x over previous
import functools

import jax
import jax.numpy as jnp
from jax import lax
from jax.experimental import pallas as pl
from jax.experimental.pallas import tpu as pltpu
from jax.experimental.pallas import tpu_sc as plsc

N = 20000
NPAD = 20480
SUBS = 16
LANES = 16
E = NPAD // SUBS
CHUNKS = E // LANES
KEEP = 100
OUTROWS = 104
REC = 2 * LANES
IMG = 1024.0
NEG = -1e9
PAD_NEG = -3e9
BIG = 3.0e38


def _build_rec(m, g, lane, x1v, y1v, x2v, y2v, arv, base):
    mv = jnp.full((LANES,), m, jnp.float32)
    gv = jnp.full((LANES,), g, jnp.float32)
    iv = jnp.full((LANES,), g.astype(jnp.int32) - base, jnp.int32)
    x1w = plsc.load_gather(x1v, [iv])
    y1w = plsc.load_gather(y1v, [iv])
    x2w = plsc.load_gather(x2v, [iv])
    y2w = plsc.load_gather(y2v, [iv])
    arw = plsc.load_gather(arv, [iv])
    return jnp.where(
        lane == 0, x1w,
        jnp.where(lane == 1, y1w,
                  jnp.where(lane == 2, x2w,
                            jnp.where(lane == 3, y2w,
                                      jnp.where(lane == 4, mv,
                                                jnp.where(lane == 5, arw, gv))))))


def _build_cands(m1, g1, m2, g2, lane, x1v, y1v, x2v, y2v, arv, base):
    M1 = jnp.max(m1)
    G1 = jnp.min(jnp.where(m1 == jnp.full((LANES,), M1, jnp.float32), g1, BIG))
    winlane = (m1 == jnp.full((LANES,), M1, jnp.float32)) & (
        g1 == jnp.full((LANES,), G1, jnp.float32))
    cv = jnp.where(winlane, m2, m1)
    cg = jnp.where(winlane, g2, g1)
    M2 = jnp.max(cv)
    G2 = jnp.min(jnp.where(cv == jnp.full((LANES,), M2, jnp.float32), cg, BIG))
    rec1 = _build_rec(M1, G1, lane, x1v, y1v, x2v, y2v, arv, base)
    rec2 = _build_rec(M2, G2, lane, x1v, y1v, x2v, y2v, arv, base)
    return rec1, rec2


def _nms_body(bx1_h, by1_h, bx2_h, by2_h, dx_h, dy_h, dw_h, dh_h, sc_h,
              out_h,
              x1v, y1v, x2v, y2v, dxv, dyv, dwv, dhv, arv, wkv,
              candv, allc, outv, shared):
    sub = lax.axis_index("s")
    core = lax.axis_index("c")
    base = sub * E
    lane = lax.broadcasted_iota(jnp.int32, (LANES,), 0)
    lanef = lane.astype(jnp.float32)
    basef = jnp.float32(E) * sub.astype(jnp.float32)
    top0 = (jnp.full((LANES,), -BIG, jnp.float32),
            jnp.zeros((LANES,), jnp.float32),
            jnp.full((LANES,), -BIG, jnp.float32),
            jnp.zeros((LANES,), jnp.float32))

    pltpu.sync_copy(bx1_h.at[pl.ds(base, E)], x1v)
    pltpu.sync_copy(by1_h.at[pl.ds(base, E)], y1v)
    pltpu.sync_copy(bx2_h.at[pl.ds(base, E)], x2v)
    pltpu.sync_copy(by2_h.at[pl.ds(base, E)], y2v)
    pltpu.sync_copy(dx_h.at[pl.ds(base, E)], dxv)
    pltpu.sync_copy(dy_h.at[pl.ds(base, E)], dyv)
    pltpu.sync_copy(dw_h.at[pl.ds(base, E)], dwv)
    pltpu.sync_copy(dh_h.at[pl.ds(base, E)], dhv)
    pltpu.sync_copy(sc_h.at[pl.ds(base, E)], wkv)

    def track2(nwk, gidf, m1, g1, m2, g2):
        gt1 = nwk > m1
        gt2 = nwk > m2
        m2n = jnp.where(gt1, m1, jnp.where(gt2, nwk, m2))
        g2n = jnp.where(gt1, g1, jnp.where(gt2, gidf, g2))
        return (jnp.where(gt1, nwk, m1), jnp.where(gt1, gidf, g1), m2n, g2n)

    def decode_chunk(j, carry):
        m1, g1, m2, g2 = carry
        sl = pl.ds(j * LANES, LANES)
        x1 = x1v[sl]
        y1 = y1v[sl]
        x2 = x2v[sl]
        y2 = y2v[sl]
        w = x2 - x1 + 1.0
        h = y2 - y1 + 1.0
        cx = x1 + 0.5 * w
        cy = y1 + 0.5 * h
        pcx = dxv[sl] * w + cx
        pcy = dyv[sl] * h + cy
        pw = jnp.exp(dwv[sl]) * w
        ph = jnp.exp(dhv[sl]) * h
        nx1 = jnp.clip(pcx - 0.5 * pw, 0.0, IMG)
        ny1 = jnp.clip(pcy - 0.5 * ph, 0.0, IMG)
        nx2 = jnp.clip(pcx + 0.5 * pw, 0.0, IMG)
        ny2 = jnp.clip(pcy + 0.5 * ph, 0.0, IMG)
        ar = jnp.maximum(nx2 - nx1, 0.0) * jnp.maximum(ny2 - ny1, 0.0)
        x1v[sl] = nx1
        y1v[sl] = ny1
        x2v[sl] = nx2
        y2v[sl] = ny2
        arv[sl] = ar
        gidf = basef + (j * LANES).astype(jnp.float32) + lanef
        return track2(wkv[sl], gidf, m1, g1, m2, g2)

    m1, g1, m2, g2 = lax.fori_loop(0, CHUNKS, decode_chunk, top0)
    rec10, rec20 = _build_cands(m1, g1, m2, g2, lane,
                                x1v, y1v, x2v, y2v, arv, base)

    def cond(carry):
        return carry[0] < KEEP

    def step(carry):
        t, e, rec1, rec2 = carry
        p = lax.bitwise_and(e, 1)
        candv[pl.ds(0, LANES)] = rec1
        candv[pl.ds(LANES, LANES)] = rec2
        pltpu.sync_copy(
            candv, shared.at[pl.ds(p * (SUBS * REC) + sub * REC, REC)])
        plsc.subcore_barrier()
        pltpu.sync_copy(shared.at[pl.ds(p * (SUBS * REC), SUBS * REC)], allc)
        vals1 = plsc.load_gather(allc, [lane * REC + 4])
        gids1 = plsc.load_gather(allc, [lane * REC + 6])
        vals2 = plsc.load_gather(allc, [lane * REC + LANES + 4])
        gids2 = plsc.load_gather(allc, [lane * REC + LANES + 6])
        M1 = jnp.max(vals1)
        tie1 = vals1 == jnp.full((LANES,), M1, jnp.float32)
        G1 = jnp.min(jnp.where(tie1, gids1, BIG))
        rowc1 = jnp.where(tie1 & (gids1 == jnp.full((LANES,), G1, jnp.float32)),
                          lane, 9999)
        r1 = jnp.min(rowc1)
        rb1 = r1 * REC
        win1 = plsc.load_gather(allc, [jnp.full((LANES,), rb1, jnp.int32) + lane])
        iw_mask = lane == jnp.full((LANES,), r1, jnp.int32)
        cv = jnp.where(iw_mask, vals2, vals1)
        cg = jnp.where(iw_mask, gids2, gids1)
        M2 = jnp.max(cv)
        tie2 = cv == jnp.full((LANES,), M2, jnp.float32)
        G2 = jnp.min(jnp.where(tie2, cg, BIG))
        rowc2 = jnp.where(tie2 & (cg == jnp.full((LANES,), G2, jnp.float32)),
                          lane, 9999)
        r2 = jnp.min(rowc2)
        rb2 = r2 * REC + jnp.where(r2 == r1, LANES, 0)
        win2 = plsc.load_gather(allc, [jnp.full((LANES,), rb2, jnp.int32) + lane])

        bx1 = plsc.load_gather(allc, [jnp.full((LANES,), rb1, jnp.int32)])
        by1 = plsc.load_gather(allc, [jnp.full((LANES,), rb1 + 1, jnp.int32)])
        bx2 = plsc.load_gather(allc, [jnp.full((LANES,), rb1 + 2, jnp.int32)])
        by2 = plsc.load_gather(allc, [jnp.full((LANES,), rb1 + 3, jnp.int32)])
        bar = plsc.load_gather(allc, [jnp.full((LANES,), rb1 + 5, jnp.int32)])
        cx1 = plsc.load_gather(allc, [jnp.full((LANES,), rb2, jnp.int32)])
        cy1 = plsc.load_gather(allc, [jnp.full((LANES,), rb2 + 1, jnp.int32)])
        cx2 = plsc.load_gather(allc, [jnp.full((LANES,), rb2 + 2, jnp.int32)])
        cy2 = plsc.load_gather(allc, [jnp.full((LANES,), rb2 + 3, jnp.int32)])
        car = plsc.load_gather(allc, [jnp.full((LANES,), rb2 + 5, jnp.int32)])

        sxx1 = jnp.maximum(bx1, cx1)
        syy1 = jnp.maximum(by1, cy1)
        sxx2 = jnp.minimum(bx2, cx2)
        syy2 = jnp.minimum(by2, cy2)
        sinter = jnp.maximum(sxx2 - sxx1, 0.0) * jnp.maximum(syy2 - syy1, 0.0)
        siou = sinter / (bar + car - sinter + 1e-9)
        specv = (~(siou > 0.5)) & (jnp.full((LANES,), M2, jnp.float32) > NEG)
        spec_i = jnp.max(jnp.where(specv, 1, 0))
        spec_b = spec_i > 0

        ok0 = (sub == 0) & (core == 0)

        @pl.when(ok0)
        def _():
            outv[pl.ds(t * LANES, LANES)] = win1

        @pl.when(ok0 & spec_b)
        def _():
            outv[pl.ds((t + 1) * LANES, LANES)] = win2

        wg1 = G1.astype(jnp.int32)
        wg2 = G2.astype(jnp.int32)

        @pl.when((wg1 >= base) & (wg1 < base + E))
        def _():
            plsc.store_scatter(wkv, [jnp.full((LANES,), wg1 - base, jnp.int32)],
                               jnp.full((LANES,), NEG, jnp.float32),
                               mask=lane == 0)

        @pl.when(spec_b & (wg2 >= base) & (wg2 < base + E))
        def _():
            plsc.store_scatter(wkv, [jnp.full((LANES,), wg2 - base, jnp.int32)],
                               jnp.full((LANES,), NEG, jnp.float32),
                               mask=lane == 0)

        cx1p = jnp.where(specv, cx1, -5000.0)
        cy1p = jnp.where(specv, cy1, -5000.0)
        cx2p = jnp.where(specv, cx2, -4000.0)
        cy2p = jnp.where(specv, cy2, -4000.0)

        def chunk(jj, carry):
            m1, g1, m2, g2 = carry
            for u in range(4):
                j = jj * 4 + u
                sl = pl.ds(j * LANES, LANES)
                x1 = x1v[sl]
                y1 = y1v[sl]
                x2 = x2v[sl]
                y2 = y2v[sl]
                ar = arv[sl]
                wk = wkv[sl]
                xx1 = jnp.maximum(bx1, x1)
                yy1 = jnp.maximum(by1, y1)
                xx2 = jnp.minimum(bx2, x2)
                yy2 = jnp.minimum(by2, y2)
                inter = jnp.maximum(xx2 - xx1, 0.0) * jnp.maximum(yy2 - yy1, 0.0)
                iou1 = inter / (bar + ar - inter + 1e-9)
                qx1 = jnp.maximum(cx1p, x1)
                qy1 = jnp.maximum(cy1p, y1)
                qx2 = jnp.minimum(cx2p, x2)
                qy2 = jnp.minimum(cy2p, y2)
                qint = jnp.maximum(qx2 - qx1, 0.0) * jnp.maximum(qy2 - qy1, 0.0)
                iou2 = qint / (car + ar - qint + 1e-9)
                sup = (iou1 > 0.5) | (iou2 > 0.5)
                nwk = jnp.where(sup, NEG, wk)
                wkv[sl] = nwk
                gidf = basef + (j * LANES).astype(jnp.float32) + lanef
                m1, g1, m2, g2 = track2(nwk, gidf, m1, g1, m2, g2)
            return (m1, g1, m2, g2)

        m1, g1, m2, g2 = lax.fori_loop(0, CHUNKS // 4, chunk, top0)
        rec1n, rec2n = _build_cands(m1, g1, m2, g2, lane,
                                    x1v, y1v, x2v, y2v, arv, base)
        return (t + 1 + spec_i, e + 1, rec1n, rec2n)

    lax.while_loop(cond, step, (jnp.int32(0), jnp.int32(0), rec10, rec20))

    @pl.when((sub == 0) & (core == 0))
    def _():
        pltpu.sync_copy(outv, out_h)


_SCRATCH = (
    [pltpu.VMEM((E,), jnp.float32)] * 10
    + [
        pltpu.VMEM((REC,), jnp.float32),
        pltpu.VMEM((SUBS * REC,), jnp.float32),
        pltpu.VMEM((OUTROWS * LANES,), jnp.float32),
        pltpu.VMEM_SHARED((2 * SUBS * REC,), jnp.float32),
    ]
)


@functools.lru_cache(maxsize=None)
def _get_nms():
    return functools.partial(
        pl.kernel,
        out_type=jax.ShapeDtypeStruct((OUTROWS * LANES,), jnp.float32),
        mesh=plsc.VectorSubcoreMesh(core_axis_name="c", subcore_axis_name="s",
                                    num_cores=1),
        compiler_params=pltpu.CompilerParams(needs_layout_passes=False),
        scratch_types=list(_SCRATCH),
    )(_nms_body)


def kernel(boxes, scores, box_deltas):
    pad = NPAD - N
    bx1 = jnp.pad(boxes[:, 0], (0, pad))
    by1 = jnp.pad(boxes[:, 1], (0, pad))
    bx2 = jnp.pad(boxes[:, 2], (0, pad))
    by2 = jnp.pad(boxes[:, 3], (0, pad))
    dx = jnp.pad(box_deltas[:, 0], (0, pad))
    dy = jnp.pad(box_deltas[:, 1], (0, pad))
    dw = jnp.pad(box_deltas[:, 2], (0, pad))
    dh = jnp.pad(box_deltas[:, 3], (0, pad))
    sc = jnp.pad(scores, (0, pad), constant_values=PAD_NEG)
    flat = _get_nms()(bx1, by1, bx2, by2, dx, dy, dw, dh, sc)
    return flat.reshape(OUTROWS, LANES)[:KEEP, :5]


# speedup vs baseline: 1.0980x; 1.0729x over previous
import functools

import jax
import jax.numpy as jnp
from jax import lax
from jax.experimental import pallas as pl
from jax.experimental.pallas import tpu as pltpu
from jax.experimental.pallas import tpu_sc as plsc

N = 20000
NPAD = 20480
SUBS = 16
LANES = 16
E = NPAD // SUBS
CHUNKS = E // LANES
KEEP = 100
OUTROWS = 104
REC = 2 * LANES
IMG = 1024.0
NEG = -1e9
PAD_NEG = -3e9
BIG = 3.0e38


def _build_rec(m, g, lane, x1v, y1v, x2v, y2v, arv, base):
    mv = jnp.full((LANES,), m, jnp.float32)
    gv = jnp.full((LANES,), g, jnp.float32)
    iv = jnp.full((LANES,), g.astype(jnp.int32) - base, jnp.int32)
    x1w = plsc.load_gather(x1v, [iv])
    y1w = plsc.load_gather(y1v, [iv])
    x2w = plsc.load_gather(x2v, [iv])
    y2w = plsc.load_gather(y2v, [iv])
    arw = plsc.load_gather(arv, [iv])
    return jnp.where(
        lane == 0, x1w,
        jnp.where(lane == 1, y1w,
                  jnp.where(lane == 2, x2w,
                            jnp.where(lane == 3, y2w,
                                      jnp.where(lane == 4, mv,
                                                jnp.where(lane == 5, arw, gv))))))


def _build_cands(m1, g1, m2, g2, lane, x1v, y1v, x2v, y2v, arv, base):
    M1 = jnp.max(m1)
    G1 = jnp.min(jnp.where(m1 == jnp.full((LANES,), M1, jnp.float32), g1, BIG))
    winlane = (m1 == jnp.full((LANES,), M1, jnp.float32)) & (
        g1 == jnp.full((LANES,), G1, jnp.float32))
    cv = jnp.where(winlane, m2, m1)
    cg = jnp.where(winlane, g2, g1)
    M2 = jnp.max(cv)
    G2 = jnp.min(jnp.where(cv == jnp.full((LANES,), M2, jnp.float32), cg, BIG))
    rec1 = _build_rec(M1, G1, lane, x1v, y1v, x2v, y2v, arv, base)
    rec2 = _build_rec(M2, G2, lane, x1v, y1v, x2v, y2v, arv, base)
    return rec1, rec2


def _nms_body(pack_h, out_h,
              x1v, y1v, x2v, y2v, dxv, dyv, dwv, dhv, arv, wkv,
              candv, allc, outv, shared):
    sub = lax.axis_index("s")
    core = lax.axis_index("c")
    base = sub * E
    lane = lax.broadcasted_iota(jnp.int32, (LANES,), 0)
    lanef = lane.astype(jnp.float32)
    basef = jnp.float32(E) * sub.astype(jnp.float32)
    top0 = (jnp.full((LANES,), -BIG, jnp.float32),
            jnp.zeros((LANES,), jnp.float32),
            jnp.full((LANES,), -BIG, jnp.float32),
            jnp.zeros((LANES,), jnp.float32))

    pltpu.sync_copy(pack_h.at[pl.ds(0 * NPAD + base, E)], x1v)
    pltpu.sync_copy(pack_h.at[pl.ds(1 * NPAD + base, E)], y1v)
    pltpu.sync_copy(pack_h.at[pl.ds(2 * NPAD + base, E)], x2v)
    pltpu.sync_copy(pack_h.at[pl.ds(3 * NPAD + base, E)], y2v)
    pltpu.sync_copy(pack_h.at[pl.ds(4 * NPAD + base, E)], dxv)
    pltpu.sync_copy(pack_h.at[pl.ds(5 * NPAD + base, E)], dyv)
    pltpu.sync_copy(pack_h.at[pl.ds(6 * NPAD + base, E)], dwv)
    pltpu.sync_copy(pack_h.at[pl.ds(7 * NPAD + base, E)], dhv)
    pltpu.sync_copy(pack_h.at[pl.ds(8 * NPAD + base, E)], wkv)

    def track2(nwk, gidf, m1, g1, m2, g2):
        gt1 = nwk > m1
        gt2 = nwk > m2
        m2n = jnp.where(gt1, m1, jnp.where(gt2, nwk, m2))
        g2n = jnp.where(gt1, g1, jnp.where(gt2, gidf, g2))
        return (jnp.where(gt1, nwk, m1), jnp.where(gt1, gidf, g1), m2n, g2n)

    def decode_chunk(j, carry):
        m1, g1, m2, g2 = carry
        sl = pl.ds(j * LANES, LANES)
        x1 = x1v[sl]
        y1 = y1v[sl]
        x2 = x2v[sl]
        y2 = y2v[sl]
        w = x2 - x1 + 1.0
        h = y2 - y1 + 1.0
        cx = x1 + 0.5 * w
        cy = y1 + 0.5 * h
        pcx = dxv[sl] * w + cx
        pcy = dyv[sl] * h + cy
        pw = jnp.exp(dwv[sl]) * w
        ph = jnp.exp(dhv[sl]) * h
        nx1 = jnp.clip(pcx - 0.5 * pw, 0.0, IMG)
        ny1 = jnp.clip(pcy - 0.5 * ph, 0.0, IMG)
        nx2 = jnp.clip(pcx + 0.5 * pw, 0.0, IMG)
        ny2 = jnp.clip(pcy + 0.5 * ph, 0.0, IMG)
        ar = jnp.maximum(nx2 - nx1, 0.0) * jnp.maximum(ny2 - ny1, 0.0)
        x1v[sl] = nx1
        y1v[sl] = ny1
        x2v[sl] = nx2
        y2v[sl] = ny2
        arv[sl] = ar
        gidf = basef + (j * LANES).astype(jnp.float32) + lanef
        return track2(wkv[sl], gidf, m1, g1, m2, g2)

    m1, g1, m2, g2 = lax.fori_loop(0, CHUNKS, decode_chunk, top0)
    rec10, rec20 = _build_cands(m1, g1, m2, g2, lane,
                                x1v, y1v, x2v, y2v, arv, base)

    def cond(carry):
        return carry[0] < KEEP

    def step(carry):
        t, e, rec1, rec2 = carry
        p = lax.bitwise_and(e, 1)
        candv[pl.ds(0, LANES)] = rec1
        candv[pl.ds(LANES, LANES)] = rec2
        pltpu.sync_copy(
            candv, shared.at[pl.ds(p * (SUBS * REC) + sub * REC, REC)])
        plsc.subcore_barrier()
        pltpu.sync_copy(shared.at[pl.ds(p * (SUBS * REC), SUBS * REC)], allc)
        vals1 = plsc.load_gather(allc, [lane * REC + 4])
        gids1 = plsc.load_gather(allc, [lane * REC + 6])
        vals2 = plsc.load_gather(allc, [lane * REC + LANES + 4])
        gids2 = plsc.load_gather(allc, [lane * REC + LANES + 6])
        M1 = jnp.max(vals1)
        tie1 = vals1 == jnp.full((LANES,), M1, jnp.float32)
        G1 = jnp.min(jnp.where(tie1, gids1, BIG))
        rowc1 = jnp.where(tie1 & (gids1 == jnp.full((LANES,), G1, jnp.float32)),
                          lane, 9999)
        r1 = jnp.min(rowc1)
        rb1 = r1 * REC
        win1 = plsc.load_gather(allc, [jnp.full((LANES,), rb1, jnp.int32) + lane])
        iw_mask = lane == jnp.full((LANES,), r1, jnp.int32)
        cv = jnp.where(iw_mask, vals2, vals1)
        cg = jnp.where(iw_mask, gids2, gids1)
        M2 = jnp.max(cv)
        tie2 = cv == jnp.full((LANES,), M2, jnp.float32)
        G2 = jnp.min(jnp.where(tie2, cg, BIG))
        rowc2 = jnp.where(tie2 & (cg == jnp.full((LANES,), G2, jnp.float32)),
                          lane, 9999)
        r2 = jnp.min(rowc2)
        rb2 = r2 * REC + jnp.where(r2 == r1, LANES, 0)
        win2 = plsc.load_gather(allc, [jnp.full((LANES,), rb2, jnp.int32) + lane])

        bx1 = plsc.load_gather(allc, [jnp.full((LANES,), rb1, jnp.int32)])
        by1 = plsc.load_gather(allc, [jnp.full((LANES,), rb1 + 1, jnp.int32)])
        bx2 = plsc.load_gather(allc, [jnp.full((LANES,), rb1 + 2, jnp.int32)])
        by2 = plsc.load_gather(allc, [jnp.full((LANES,), rb1 + 3, jnp.int32)])
        bar = plsc.load_gather(allc, [jnp.full((LANES,), rb1 + 5, jnp.int32)])
        cx1 = plsc.load_gather(allc, [jnp.full((LANES,), rb2, jnp.int32)])
        cy1 = plsc.load_gather(allc, [jnp.full((LANES,), rb2 + 1, jnp.int32)])
        cx2 = plsc.load_gather(allc, [jnp.full((LANES,), rb2 + 2, jnp.int32)])
        cy2 = plsc.load_gather(allc, [jnp.full((LANES,), rb2 + 3, jnp.int32)])
        car = plsc.load_gather(allc, [jnp.full((LANES,), rb2 + 5, jnp.int32)])

        sxx1 = jnp.maximum(bx1, cx1)
        syy1 = jnp.maximum(by1, cy1)
        sxx2 = jnp.minimum(bx2, cx2)
        syy2 = jnp.minimum(by2, cy2)
        sinter = jnp.maximum(sxx2 - sxx1, 0.0) * jnp.maximum(syy2 - syy1, 0.0)
        siou = sinter / (bar + car - sinter + 1e-9)
        specv = (~(siou > 0.5)) & (jnp.full((LANES,), M2, jnp.float32) > NEG)
        spec_i = jnp.max(jnp.where(specv, 1, 0))
        spec_b = spec_i > 0

        ok0 = (sub == 0) & (core == 0)

        @pl.when(ok0)
        def _():
            outv[pl.ds(t * LANES, LANES)] = win1

        @pl.when(ok0 & spec_b)
        def _():
            outv[pl.ds((t + 1) * LANES, LANES)] = win2

        wg1 = G1.astype(jnp.int32)
        wg2 = G2.astype(jnp.int32)

        @pl.when((wg1 >= base) & (wg1 < base + E))
        def _():
            plsc.store_scatter(wkv, [jnp.full((LANES,), wg1 - base, jnp.int32)],
                               jnp.full((LANES,), NEG, jnp.float32),
                               mask=lane == 0)

        @pl.when(spec_b & (wg2 >= base) & (wg2 < base + E))
        def _():
            plsc.store_scatter(wkv, [jnp.full((LANES,), wg2 - base, jnp.int32)],
                               jnp.full((LANES,), NEG, jnp.float32),
                               mask=lane == 0)

        cx1p = jnp.where(specv, cx1, -5000.0)
        cy1p = jnp.where(specv, cy1, -5000.0)
        cx2p = jnp.where(specv, cx2, -4000.0)
        cy2p = jnp.where(specv, cy2, -4000.0)

        def chunk(jj, carry):
            m1, g1, m2, g2 = carry
            for u in range(4):
                j = jj * 4 + u
                sl = pl.ds(j * LANES, LANES)
                x1 = x1v[sl]
                y1 = y1v[sl]
                x2 = x2v[sl]
                y2 = y2v[sl]
                ar = arv[sl]
                wk = wkv[sl]
                xx1 = jnp.maximum(bx1, x1)
                yy1 = jnp.maximum(by1, y1)
                xx2 = jnp.minimum(bx2, x2)
                yy2 = jnp.minimum(by2, y2)
                inter = jnp.maximum(xx2 - xx1, 0.0) * jnp.maximum(yy2 - yy1, 0.0)
                iou1 = inter / (bar + ar - inter + 1e-9)
                qx1 = jnp.maximum(cx1p, x1)
                qy1 = jnp.maximum(cy1p, y1)
                qx2 = jnp.minimum(cx2p, x2)
                qy2 = jnp.minimum(cy2p, y2)
                qint = jnp.maximum(qx2 - qx1, 0.0) * jnp.maximum(qy2 - qy1, 0.0)
                iou2 = qint / (car + ar - qint + 1e-9)
                sup = (iou1 > 0.5) | (iou2 > 0.5)
                nwk = jnp.where(sup, NEG, wk)
                wkv[sl] = nwk
                gidf = basef + (j * LANES).astype(jnp.float32) + lanef
                m1, g1, m2, g2 = track2(nwk, gidf, m1, g1, m2, g2)
            return (m1, g1, m2, g2)

        m1, g1, m2, g2 = lax.fori_loop(0, CHUNKS // 4, chunk, top0)
        rec1n, rec2n = _build_cands(m1, g1, m2, g2, lane,
                                    x1v, y1v, x2v, y2v, arv, base)
        return (t + 1 + spec_i, e + 1, rec1n, rec2n)

    lax.while_loop(cond, step, (jnp.int32(0), jnp.int32(0), rec10, rec20))

    @pl.when((sub == 0) & (core == 0))
    def _():
        pltpu.sync_copy(outv, out_h)


_SCRATCH = (
    [pltpu.VMEM((E,), jnp.float32)] * 10
    + [
        pltpu.VMEM((REC,), jnp.float32),
        pltpu.VMEM((SUBS * REC,), jnp.float32),
        pltpu.VMEM((OUTROWS * LANES,), jnp.float32),
        pltpu.VMEM_SHARED((2 * SUBS * REC,), jnp.float32),
    ]
)


@functools.lru_cache(maxsize=None)
def _get_nms():
    return functools.partial(
        pl.kernel,
        out_type=jax.ShapeDtypeStruct((OUTROWS * LANES,), jnp.float32),
        mesh=plsc.VectorSubcoreMesh(core_axis_name="c", subcore_axis_name="s",
                                    num_cores=1),
        compiler_params=pltpu.CompilerParams(needs_layout_passes=False),
        scratch_types=list(_SCRATCH),
    )(_nms_body)


def kernel(boxes, scores, box_deltas):
    cols = jnp.concatenate([boxes.T, box_deltas.T, scores[None, :]], axis=0)
    pack = jnp.pad(cols, ((0, 0), (0, NPAD - N)),
                   constant_values=PAD_NEG).reshape(-1)
    flat = _get_nms()(pack)
    return flat.reshape(OUTROWS, LANES)[:KEEP, :5]


# speedup vs baseline: 1.0982x; 1.0002x over previous
import functools

import jax
import jax.numpy as jnp
from jax import lax
from jax.experimental import pallas as pl
from jax.experimental.pallas import tpu as pltpu
from jax.experimental.pallas import tpu_sc as plsc

N = 20000
NPAD = 20480
SUBS = 16
LANES = 16
E = NPAD // SUBS
CHUNKS = E // LANES
KEEP = 100
OUTROWS = 104
REC = 2 * LANES
IMG = 1024.0
NEG = -1e9
PAD_NEG = -3e9
BIG = 3.0e38


def _build_rec(m, g, lane, x1v, y1v, x2v, y2v, arv, base):
    mv = jnp.full((LANES,), m, jnp.float32)
    gv = jnp.full((LANES,), g, jnp.float32)
    iv = jnp.full((LANES,), g.astype(jnp.int32) - base, jnp.int32)
    x1w = plsc.load_gather(x1v, [iv])
    y1w = plsc.load_gather(y1v, [iv])
    x2w = plsc.load_gather(x2v, [iv])
    y2w = plsc.load_gather(y2v, [iv])
    arw = plsc.load_gather(arv, [iv])
    return jnp.where(
        lane == 0, x1w,
        jnp.where(lane == 1, y1w,
                  jnp.where(lane == 2, x2w,
                            jnp.where(lane == 3, y2w,
                                      jnp.where(lane == 4, mv,
                                                jnp.where(lane == 5, arw, gv))))))


def _build_cands(m1, g1, m2, g2, lane, x1v, y1v, x2v, y2v, arv, base):
    M1 = jnp.max(m1)
    G1 = jnp.min(jnp.where(m1 == jnp.full((LANES,), M1, jnp.float32), g1, BIG))
    winlane = (m1 == jnp.full((LANES,), M1, jnp.float32)) & (
        g1 == jnp.full((LANES,), G1, jnp.float32))
    cv = jnp.where(winlane, m2, m1)
    cg = jnp.where(winlane, g2, g1)
    M2 = jnp.max(cv)
    G2 = jnp.min(jnp.where(cv == jnp.full((LANES,), M2, jnp.float32), cg, BIG))
    rec1 = _build_rec(M1, G1, lane, x1v, y1v, x2v, y2v, arv, base)
    rec2 = _build_rec(M2, G2, lane, x1v, y1v, x2v, y2v, arv, base)
    return rec1, rec2


def _nms_body(pack_h, out_h,
              x1v, y1v, x2v, y2v, dxv, dyv, dwv, dhv, arv, wkv,
              candv, allc, outv, shared):
    sub = lax.axis_index("s")
    core = lax.axis_index("c")
    base = sub * E
    lane = lax.broadcasted_iota(jnp.int32, (LANES,), 0)
    lanef = lane.astype(jnp.float32)
    basef = jnp.float32(E) * sub.astype(jnp.float32)
    top0 = (jnp.full((LANES,), -BIG, jnp.float32),
            jnp.zeros((LANES,), jnp.float32),
            jnp.full((LANES,), -BIG, jnp.float32),
            jnp.zeros((LANES,), jnp.float32))

    pltpu.sync_copy(pack_h.at[pl.ds(0 * NPAD + base, E)], x1v)
    pltpu.sync_copy(pack_h.at[pl.ds(1 * NPAD + base, E)], y1v)
    pltpu.sync_copy(pack_h.at[pl.ds(2 * NPAD + base, E)], x2v)
    pltpu.sync_copy(pack_h.at[pl.ds(3 * NPAD + base, E)], y2v)
    pltpu.sync_copy(pack_h.at[pl.ds(4 * NPAD + base, E)], dxv)
    pltpu.sync_copy(pack_h.at[pl.ds(5 * NPAD + base, E)], dyv)
    pltpu.sync_copy(pack_h.at[pl.ds(6 * NPAD + base, E)], dwv)
    pltpu.sync_copy(pack_h.at[pl.ds(7 * NPAD + base, E)], dhv)
    pltpu.sync_copy(pack_h.at[pl.ds(8 * NPAD + base, E)], wkv)

    def track2(nwk, gidf, m1, g1, m2, g2):
        gt1 = nwk > m1
        gt2 = nwk > m2
        m2n = jnp.where(gt1, m1, jnp.where(gt2, nwk, m2))
        g2n = jnp.where(gt1, g1, jnp.where(gt2, gidf, g2))
        return (jnp.where(gt1, nwk, m1), jnp.where(gt1, gidf, g1), m2n, g2n)

    def decode_chunk(jj, carry):
        m1, g1, m2, g2 = carry
        for u in range(4):
            j = jj * 4 + u
            sl = pl.ds(j * LANES, LANES)
            x1 = x1v[sl]
            y1 = y1v[sl]
            x2 = x2v[sl]
            y2 = y2v[sl]
            w = x2 - x1 + 1.0
            h = y2 - y1 + 1.0
            cx = x1 + 0.5 * w
            cy = y1 + 0.5 * h
            pcx = dxv[sl] * w + cx
            pcy = dyv[sl] * h + cy
            pw = jnp.exp(dwv[sl]) * w
            ph = jnp.exp(dhv[sl]) * h
            nx1 = jnp.clip(pcx - 0.5 * pw, 0.0, IMG)
            ny1 = jnp.clip(pcy - 0.5 * ph, 0.0, IMG)
            nx2 = jnp.clip(pcx + 0.5 * pw, 0.0, IMG)
            ny2 = jnp.clip(pcy + 0.5 * ph, 0.0, IMG)
            ar = jnp.maximum(nx2 - nx1, 0.0) * jnp.maximum(ny2 - ny1, 0.0)
            x1v[sl] = nx1
            y1v[sl] = ny1
            x2v[sl] = nx2
            y2v[sl] = ny2
            arv[sl] = ar
            gidf = basef + (j * LANES).astype(jnp.float32) + lanef
            m1, g1, m2, g2 = track2(wkv[sl], gidf, m1, g1, m2, g2)
        return (m1, g1, m2, g2)

    m1, g1, m2, g2 = lax.fori_loop(0, CHUNKS // 4, decode_chunk, top0)
    rec10, rec20 = _build_cands(m1, g1, m2, g2, lane,
                                x1v, y1v, x2v, y2v, arv, base)

    def cond(carry):
        return carry[0] < KEEP

    def step(carry):
        t, e, rec1, rec2 = carry
        p = lax.bitwise_and(e, 1)
        candv[pl.ds(0, LANES)] = rec1
        candv[pl.ds(LANES, LANES)] = rec2
        pltpu.sync_copy(
            candv, shared.at[pl.ds(p * (SUBS * REC) + sub * REC, REC)])
        plsc.subcore_barrier()
        pltpu.sync_copy(shared.at[pl.ds(p * (SUBS * REC), SUBS * REC)], allc)
        vals1 = plsc.load_gather(allc, [lane * REC + 4])
        gids1 = plsc.load_gather(allc, [lane * REC + 6])
        vals2 = plsc.load_gather(allc, [lane * REC + LANES + 4])
        gids2 = plsc.load_gather(allc, [lane * REC + LANES + 6])
        M1 = jnp.max(vals1)
        tie1 = vals1 == jnp.full((LANES,), M1, jnp.float32)
        G1 = jnp.min(jnp.where(tie1, gids1, BIG))
        rowc1 = jnp.where(tie1 & (gids1 == jnp.full((LANES,), G1, jnp.float32)),
                          lane, 9999)
        r1 = jnp.min(rowc1)
        rb1 = r1 * REC
        win1 = plsc.load_gather(allc, [jnp.full((LANES,), rb1, jnp.int32) + lane])
        iw_mask = lane == jnp.full((LANES,), r1, jnp.int32)
        cv = jnp.where(iw_mask, vals2, vals1)
        cg = jnp.where(iw_mask, gids2, gids1)
        M2 = jnp.max(cv)
        tie2 = cv == jnp.full((LANES,), M2, jnp.float32)
        G2 = jnp.min(jnp.where(tie2, cg, BIG))
        rowc2 = jnp.where(tie2 & (cg == jnp.full((LANES,), G2, jnp.float32)),
                          lane, 9999)
        r2 = jnp.min(rowc2)
        rb2 = r2 * REC + jnp.where(r2 == r1, LANES, 0)
        win2 = plsc.load_gather(allc, [jnp.full((LANES,), rb2, jnp.int32) + lane])

        bx1 = plsc.load_gather(allc, [jnp.full((LANES,), rb1, jnp.int32)])
        by1 = plsc.load_gather(allc, [jnp.full((LANES,), rb1 + 1, jnp.int32)])
        bx2 = plsc.load_gather(allc, [jnp.full((LANES,), rb1 + 2, jnp.int32)])
        by2 = plsc.load_gather(allc, [jnp.full((LANES,), rb1 + 3, jnp.int32)])
        bar = plsc.load_gather(allc, [jnp.full((LANES,), rb1 + 5, jnp.int32)])
        cx1 = plsc.load_gather(allc, [jnp.full((LANES,), rb2, jnp.int32)])
        cy1 = plsc.load_gather(allc, [jnp.full((LANES,), rb2 + 1, jnp.int32)])
        cx2 = plsc.load_gather(allc, [jnp.full((LANES,), rb2 + 2, jnp.int32)])
        cy2 = plsc.load_gather(allc, [jnp.full((LANES,), rb2 + 3, jnp.int32)])
        car = plsc.load_gather(allc, [jnp.full((LANES,), rb2 + 5, jnp.int32)])

        sxx1 = jnp.maximum(bx1, cx1)
        syy1 = jnp.maximum(by1, cy1)
        sxx2 = jnp.minimum(bx2, cx2)
        syy2 = jnp.minimum(by2, cy2)
        sinter = jnp.maximum(sxx2 - sxx1, 0.0) * jnp.maximum(syy2 - syy1, 0.0)
        siou = sinter / (bar + car - sinter + 1e-9)
        specv = (~(siou > 0.5)) & (jnp.full((LANES,), M2, jnp.float32) > NEG)
        spec_i = jnp.max(jnp.where(specv, 1, 0))
        spec_b = spec_i > 0

        ok0 = (sub == 0) & (core == 0)

        @pl.when(ok0)
        def _():
            outv[pl.ds(t * LANES, LANES)] = win1

        @pl.when(ok0 & spec_b)
        def _():
            outv[pl.ds((t + 1) * LANES, LANES)] = win2

        wg1 = G1.astype(jnp.int32)
        wg2 = G2.astype(jnp.int32)

        @pl.when((wg1 >= base) & (wg1 < base + E))
        def _():
            plsc.store_scatter(wkv, [jnp.full((LANES,), wg1 - base, jnp.int32)],
                               jnp.full((LANES,), NEG, jnp.float32),
                               mask=lane == 0)

        @pl.when(spec_b & (wg2 >= base) & (wg2 < base + E))
        def _():
            plsc.store_scatter(wkv, [jnp.full((LANES,), wg2 - base, jnp.int32)],
                               jnp.full((LANES,), NEG, jnp.float32),
                               mask=lane == 0)

        cx1p = jnp.where(specv, cx1, -5000.0)
        cy1p = jnp.where(specv, cy1, -5000.0)
        cx2p = jnp.where(specv, cx2, -4000.0)
        cy2p = jnp.where(specv, cy2, -4000.0)

        def chunk(jj, carry):
            m1, g1, m2, g2 = carry
            for u in range(4):
                j = jj * 4 + u
                sl = pl.ds(j * LANES, LANES)
                x1 = x1v[sl]
                y1 = y1v[sl]
                x2 = x2v[sl]
                y2 = y2v[sl]
                ar = arv[sl]
                wk = wkv[sl]
                xx1 = jnp.maximum(bx1, x1)
                yy1 = jnp.maximum(by1, y1)
                xx2 = jnp.minimum(bx2, x2)
                yy2 = jnp.minimum(by2, y2)
                inter = jnp.maximum(xx2 - xx1, 0.0) * jnp.maximum(yy2 - yy1, 0.0)
                iou1 = inter / (bar + ar - inter + 1e-9)
                qx1 = jnp.maximum(cx1p, x1)
                qy1 = jnp.maximum(cy1p, y1)
                qx2 = jnp.minimum(cx2p, x2)
                qy2 = jnp.minimum(cy2p, y2)
                qint = jnp.maximum(qx2 - qx1, 0.0) * jnp.maximum(qy2 - qy1, 0.0)
                iou2 = qint / (car + ar - qint + 1e-9)
                sup = (iou1 > 0.5) | (iou2 > 0.5)
                nwk = jnp.where(sup, NEG, wk)
                wkv[sl] = nwk
                gidf = basef + (j * LANES).astype(jnp.float32) + lanef
                m1, g1, m2, g2 = track2(nwk, gidf, m1, g1, m2, g2)
            return (m1, g1, m2, g2)

        m1, g1, m2, g2 = lax.fori_loop(0, CHUNKS // 4, chunk, top0)
        rec1n, rec2n = _build_cands(m1, g1, m2, g2, lane,
                                    x1v, y1v, x2v, y2v, arv, base)
        return (t + 1 + spec_i, e + 1, rec1n, rec2n)

    lax.while_loop(cond, step, (jnp.int32(0), jnp.int32(0), rec10, rec20))

    @pl.when((sub == 0) & (core == 0))
    def _():
        pltpu.sync_copy(outv, out_h)


_SCRATCH = (
    [pltpu.VMEM((E,), jnp.float32)] * 10
    + [
        pltpu.VMEM((REC,), jnp.float32),
        pltpu.VMEM((SUBS * REC,), jnp.float32),
        pltpu.VMEM((OUTROWS * LANES,), jnp.float32),
        pltpu.VMEM_SHARED((2 * SUBS * REC,), jnp.float32),
    ]
)


@functools.lru_cache(maxsize=None)
def _get_nms():
    return functools.partial(
        pl.kernel,
        out_type=jax.ShapeDtypeStruct((OUTROWS * LANES,), jnp.float32),
        mesh=plsc.VectorSubcoreMesh(core_axis_name="c", subcore_axis_name="s",
                                    num_cores=1),
        compiler_params=pltpu.CompilerParams(needs_layout_passes=False),
        scratch_types=list(_SCRATCH),
    )(_nms_body)


def kernel(boxes, scores, box_deltas):
    cols = jnp.concatenate([boxes.T, box_deltas.T, scores[None, :]], axis=0)
    pack = jnp.pad(cols, ((0, 0), (0, NPAD - N)),
                   constant_values=PAD_NEG).reshape(-1)
    flat = _get_nms()(pack)
    return flat.reshape(OUTROWS, LANES)[:KEEP, :5]
